# baseline (speedup 1.0000x reference)
import functools

import jax
import jax.numpy as jnp
from jax.experimental import pallas as pl
from jax.experimental.pallas import tpu as pltpu
from jax.experimental.pallas import tpu_sc as plsc

_K = 1024
_D = 64
_BLK = 1024
_C = 1


def _vq_body(x_ref, cb_ref, xsq_ref, csq_ref, idx_ref):
    xb = x_ref[...]
    cb = cb_ref[...]
    mm = jax.lax.dot_general(xb, cb, (((1,), (1,)), ((), ())),
                             preferred_element_type=jnp.float32)
    x_sq = xsq_ref[...]
    c_sq = csq_ref[...]
    d2 = x_sq + c_sq - 2.0 * mm
    d2c = jnp.maximum(d2, 0.0)
    dist = jnp.where(d2c == 0.0, 0.0, d2c * jax.lax.rsqrt(d2c))
    m = jnp.min(dist, axis=1, keepdims=True)
    lane = jax.lax.broadcasted_iota(jnp.int32, (_BLK, _K), 1)
    idx = jnp.min(jnp.where(dist == m, lane, _K), axis=1).astype(jnp.int32)
    idx_ref[...] = idx


def _vq_indices(flat_x, codebook, xsq, csq, start, rows):
    c0 = start // _BLK
    return pl.pallas_call(
        _vq_body,
        grid=(rows // _BLK,),
        in_specs=[
            pl.BlockSpec((_BLK, _D), lambda i: (c0 + i, 0)),
            pl.BlockSpec((_K, _D), lambda i: (0, 0)),
            pl.BlockSpec((_BLK, 1), lambda i: (c0 + i, 0)),
            pl.BlockSpec((1, _K), lambda i: (0, 0)),
        ],
        out_specs=pl.BlockSpec((_BLK,), lambda i: (i,)),
        out_shape=jax.ShapeDtypeStruct((rows,), jnp.int32),
    )(flat_x, codebook, xsq, csq)


def _sc_gather(codebook_pad, idx_flat):
    n = idx_flat.shape[0]
    mesh = plsc.VectorSubcoreMesh(core_axis_name="c", subcore_axis_name="s")
    nc = mesh.num_cores
    nw = nc * mesh.num_subcores
    b_per_w = n // nw
    pieces = 1
    while b_per_w // pieces > 1000 or b_per_w % pieces:
        pieces += 1
    piece = b_per_w // pieces

    @functools.partial(
        pl.kernel,
        out_type=jax.ShapeDtypeStruct((n, 128), jnp.float32),
        mesh=mesh,
        scratch_types=[
            pltpu.VMEM((piece,), jnp.int32),
            pltpu.VMEM((piece, 128), jnp.float32),
            pltpu.SemaphoreType.DMA,
        ],
    )
    def gk(cb_hbm, idx_hbm, out_hbm, idx_v, rows_v, sem):
        wid = jax.lax.axis_index("s") * nc + jax.lax.axis_index("c")
        for h in range(pieces):
            base = wid * b_per_w + h * piece
            pltpu.sync_copy(idx_hbm.at[pl.ds(base, piece)], idx_v)
            pltpu.async_copy(cb_hbm.at[idx_v], rows_v, sem).wait()
            pltpu.sync_copy(rows_v, out_hbm.at[pl.ds(base, piece)])

    return gk(codebook_pad, idx_flat)


def kernel(x, codebook):
    input_shape = x.shape
    flat_x = x.reshape(-1, codebook.shape[1])
    n = flat_x.shape[0]
    xsq = jnp.sum(flat_x * flat_x, axis=1, keepdims=True)
    csq = jnp.sum(codebook * codebook, axis=1)[None, :]
    cb_pad = jnp.pad(codebook, ((0, 0), (0, 128 - _D)))
    chunk = n // _C
    idxs, qs = [], []
    for c in range(_C):
        idx_c = _vq_indices(flat_x, codebook, xsq, csq, c * chunk, chunk)
        qs.append(_sc_gather(cb_pad, idx_c))
        idxs.append(idx_c)
    idx = jnp.concatenate(idxs)
    q = jnp.concatenate(qs)[:, :_D]
    return idx.reshape(input_shape[:-1]), q.reshape(input_shape)


# speedup vs baseline: 1.1433x; 1.1433x over previous
import functools

import jax
import jax.numpy as jnp
from jax.experimental import pallas as pl
from jax.experimental.pallas import tpu as pltpu
from jax.experimental.pallas import tpu_sc as plsc

_K = 1024
_D = 64
_BLK = 4096
_C = 1


def _vq_body(x_ref, cb_ref, xsq_ref, csq_ref, idx_ref):
    xb = x_ref[...]
    cb = cb_ref[...]
    mm = jax.lax.dot_general(xb, cb, (((1,), (1,)), ((), ())),
                             preferred_element_type=jnp.float32)
    x_sq = xsq_ref[...]
    c_sq = csq_ref[...]
    d2 = x_sq + c_sq - 2.0 * mm
    d2c = jnp.maximum(d2, 0.0)
    dist = jnp.where(d2c == 0.0, 0.0, d2c * jax.lax.rsqrt(d2c))
    m = jnp.min(dist, axis=1, keepdims=True)
    lane = jax.lax.broadcasted_iota(jnp.int32, (_BLK, _K), 1)
    idx = jnp.min(jnp.where(dist == m, lane, _K), axis=1).astype(jnp.int32)
    idx_ref[...] = idx


def _vq_indices(flat_x, codebook, xsq, csq, start, rows):
    c0 = start // _BLK
    return pl.pallas_call(
        _vq_body,
        grid=(rows // _BLK,),
        in_specs=[
            pl.BlockSpec((_BLK, _D), lambda i: (c0 + i, 0)),
            pl.BlockSpec((_K, _D), lambda i: (0, 0)),
            pl.BlockSpec((_BLK, 1), lambda i: (c0 + i, 0)),
            pl.BlockSpec((1, _K), lambda i: (0, 0)),
        ],
        out_specs=pl.BlockSpec((_BLK,), lambda i: (i,)),
        out_shape=jax.ShapeDtypeStruct((rows,), jnp.int32),
    )(flat_x, codebook, xsq, csq)


def _sc_gather(codebook_pad, idx_flat):
    n = idx_flat.shape[0]
    mesh = plsc.VectorSubcoreMesh(core_axis_name="c", subcore_axis_name="s")
    nc = mesh.num_cores
    nw = nc * mesh.num_subcores
    b_per_w = n // nw
    pieces = 1
    while b_per_w // pieces > 1000 or b_per_w % pieces:
        pieces += 1
    piece = b_per_w // pieces

    @functools.partial(
        pl.kernel,
        out_type=jax.ShapeDtypeStruct((n, 128), jnp.float32),
        mesh=mesh,
        scratch_types=[
            pltpu.VMEM((piece,), jnp.int32),
            pltpu.VMEM((piece, 128), jnp.float32),
            pltpu.SemaphoreType.DMA,
        ],
    )
    def gk(cb_hbm, idx_hbm, out_hbm, idx_v, rows_v, sem):
        wid = jax.lax.axis_index("s") * nc + jax.lax.axis_index("c")
        for h in range(pieces):
            base = wid * b_per_w + h * piece
            pltpu.sync_copy(idx_hbm.at[pl.ds(base, piece)], idx_v)
            pltpu.async_copy(cb_hbm.at[idx_v], rows_v, sem).wait()
            pltpu.sync_copy(rows_v, out_hbm.at[pl.ds(base, piece)])

    return gk(codebook_pad, idx_flat)


def kernel(x, codebook):
    input_shape = x.shape
    flat_x = x.reshape(-1, codebook.shape[1])
    n = flat_x.shape[0]
    xsq = jnp.sum(flat_x * flat_x, axis=1, keepdims=True)
    csq = jnp.sum(codebook * codebook, axis=1)[None, :]
    cb_pad = jnp.pad(codebook, ((0, 0), (0, 128 - _D)))
    chunk = n // _C
    idxs, qs = [], []
    for c in range(_C):
        idx_c = _vq_indices(flat_x, codebook, xsq, csq, c * chunk, chunk)
        qs.append(_sc_gather(cb_pad, idx_c))
        idxs.append(idx_c)
    idx = jnp.concatenate(idxs)
    q = jnp.concatenate(qs)[:, :_D]
    return idx.reshape(input_shape[:-1]), q.reshape(input_shape)


# speedup vs baseline: 1.1562x; 1.0113x over previous
import functools

import jax
import jax.numpy as jnp
from jax.experimental import pallas as pl
from jax.experimental.pallas import tpu as pltpu
from jax.experimental.pallas import tpu_sc as plsc

_K = 1024
_D = 64
_BLK = 6144
_C = 1


def _vq_body(x_ref, cb_ref, xsq_ref, csq_ref, idx_ref):
    xb = x_ref[...]
    cb = cb_ref[...]
    mm = jax.lax.dot_general(xb, cb, (((1,), (1,)), ((), ())),
                             preferred_element_type=jnp.float32)
    x_sq = xsq_ref[...]
    c_sq = csq_ref[...]
    d2 = x_sq + c_sq - 2.0 * mm
    d2c = jnp.maximum(d2, 0.0)
    dist = jnp.where(d2c == 0.0, 0.0, d2c * jax.lax.rsqrt(d2c))
    m = jnp.min(dist, axis=1, keepdims=True)
    lane = jax.lax.broadcasted_iota(jnp.int32, (_BLK, _K), 1)
    idx = jnp.min(jnp.where(dist == m, lane, _K), axis=1).astype(jnp.int32)
    idx_ref[...] = idx


def _vq_indices(flat_x, codebook, xsq, csq, start, rows):
    c0 = start // _BLK
    return pl.pallas_call(
        _vq_body,
        grid=(rows // _BLK,),
        in_specs=[
            pl.BlockSpec((_BLK, _D), lambda i: (c0 + i, 0)),
            pl.BlockSpec((_K, _D), lambda i: (0, 0)),
            pl.BlockSpec((_BLK, 1), lambda i: (c0 + i, 0)),
            pl.BlockSpec((1, _K), lambda i: (0, 0)),
        ],
        out_specs=pl.BlockSpec((_BLK,), lambda i: (i,)),
        out_shape=jax.ShapeDtypeStruct((rows,), jnp.int32),
    )(flat_x, codebook, xsq, csq)


def _sc_gather(codebook_pad, idx_flat):
    n = idx_flat.shape[0]
    mesh = plsc.VectorSubcoreMesh(core_axis_name="c", subcore_axis_name="s")
    nc = mesh.num_cores
    nw = nc * mesh.num_subcores
    b_per_w = n // nw
    pieces = 1
    while b_per_w // pieces > 1000 or b_per_w % pieces:
        pieces += 1
    piece = b_per_w // pieces

    @functools.partial(
        pl.kernel,
        out_type=jax.ShapeDtypeStruct((n, 128), jnp.float32),
        mesh=mesh,
        scratch_types=[
            pltpu.VMEM((piece,), jnp.int32),
            pltpu.VMEM((piece, 128), jnp.float32),
            pltpu.SemaphoreType.DMA,
        ],
    )
    def gk(cb_hbm, idx_hbm, out_hbm, idx_v, rows_v, sem):
        wid = jax.lax.axis_index("s") * nc + jax.lax.axis_index("c")
        for h in range(pieces):
            base = wid * b_per_w + h * piece
            pltpu.sync_copy(idx_hbm.at[pl.ds(base, piece)], idx_v)
            pltpu.async_copy(cb_hbm.at[idx_v], rows_v, sem).wait()
            pltpu.sync_copy(rows_v, out_hbm.at[pl.ds(base, piece)])

    return gk(codebook_pad, idx_flat)


def kernel(x, codebook):
    input_shape = x.shape
    flat_x = x.reshape(-1, codebook.shape[1])
    n = flat_x.shape[0]
    xsq = jnp.sum(flat_x * flat_x, axis=1, keepdims=True)
    csq = jnp.sum(codebook * codebook, axis=1)[None, :]
    cb_pad = jnp.pad(codebook, ((0, 0), (0, 128 - _D)))
    chunk = n // _C
    idxs, qs = [], []
    for c in range(_C):
        idx_c = _vq_indices(flat_x, codebook, xsq, csq, c * chunk, chunk)
        qs.append(_sc_gather(cb_pad, idx_c))
        idxs.append(idx_c)
    idx = jnp.concatenate(idxs)
    q = jnp.concatenate(qs)[:, :_D]
    return idx.reshape(input_shape[:-1]), q.reshape(input_shape)
